# baseline (speedup 1.0000x reference)
import functools

import jax
import jax.numpy as jnp
from jax import lax
from jax.experimental import pallas as pl
from jax.experimental.pallas import tpu as pltpu
from jax.experimental.pallas import tpu_sc as plsc

_N1 = 16384
_N2 = 16384
_D = 128
_NB = 16
_HID = 128

_BM = 2048
_BQ = 256
_BK = 512
_NQB = _N2 // _BQ


def _k1_body(bounds_ref, x2t_ref, x1t_ref, xf_ref, w1_ref,
             b1_ref, u_ref, col_ref):
    q = pl.program_id(0)

    u = jnp.dot(xf_ref[...], w1_ref[0:_D, :],
                preferred_element_type=jnp.float32)
    t3 = x1t_ref[0:3, pl.ds(q * _BQ, _BQ)]
    u += lax.dot_general(t3, w1_ref[_D:, :], (((0,), (0,)), ((), ())),
                         preferred_element_type=jnp.float32)
    u_ref[...] = u + b1_ref[...]

    lo = bounds_ref[q, 0]
    hi = bounds_ref[q, 1]
    kb0 = lo // _BK
    kb1 = (hi + _BK - 1) // _BK
    t4 = x2t_ref[...]
    m2x = jnp.swapaxes(-2.0 * t4[0:1, :], 0, 1)
    m2y = jnp.swapaxes(-2.0 * t4[1:2, :], 0, 1)
    m2z = jnp.swapaxes(-2.0 * t4[2:3, :], 0, 1)
    qb = jnp.swapaxes(t4[3:4, :], 0, 1)
    inf = jnp.float32(jnp.inf)
    lanef = lax.broadcasted_iota(jnp.int32, (_BQ, _BK), 1).astype(jnp.float32)

    def one_tile(kb, valid, carry):
        bd, bl, bk = carry
        off = kb * _BK
        px = x1t_ref[0:1, pl.ds(off, _BK)]
        py = x1t_ref[1:2, pl.ds(off, _BK)]
        pz = x1t_ref[2:3, pl.ds(off, _BK)]
        ppt = x1t_ref[3:4, pl.ds(off, _BK)]
        tb = x1t_ref[4:5, pl.ds(off, _BK)]
        d = ppt + px * m2x + py * m2y + pz * m2z
        d = jnp.where(qb == tb, d, inf)
        tmin = jnp.min(d, axis=1, keepdims=True)
        cand = jnp.where(d == tmin, lanef, jnp.float32(1e9))
        targ = jnp.min(cand, axis=1, keepdims=True)
        upd = (tmin < bd) & valid
        kbf = jnp.full((_BQ, 1), kb, jnp.float32)
        return (jnp.where(upd, tmin, bd), jnp.where(upd, targ, bl),
                jnp.where(upd, kbf, bk))

    last_kb = jnp.int32(_N1 // _BK - 1)

    def tile4(i, carry):
        kb = kb0 + 4 * i
        carry = one_tile(kb, True, carry)
        for j in (1, 2, 3):
            kbj = jnp.minimum(kb + j, last_kb)
            carry = one_tile(kbj, kb + j < kb1, carry)
        return carry

    bd0 = jnp.full((_BQ, 1), inf, jnp.float32)
    bl0 = jnp.zeros((_BQ, 1), jnp.float32)
    bk0 = jnp.zeros((_BQ, 1), jnp.float32)
    trips = (kb1 - kb0 + 3) // 4
    _, bl, bk = lax.fori_loop(0, trips, tile4, (bd0, bl0, bk0))
    col = (bk * float(_BK) + bl).astype(jnp.int32)
    col_ref[...] = col.T.reshape(1, 1, _BQ)


def _mlp_body(g_ref, x2t_ref, w1_ref, w2_ref, b2_ref, o_ref):
    i = pl.program_id(0)
    t3 = x2t_ref[0:3, pl.ds(i * _BM, _BM)]
    v = lax.dot_general(t3, w1_ref[_D:, :], (((0,), (0,)), ((), ())),
                        preferred_element_type=jnp.float32)
    h1 = jnp.maximum(g_ref[...] - v, 0.0)
    h2 = jnp.dot(h1, w2_ref[...], preferred_element_type=jnp.float32) + b2_ref[...]
    o_ref[...] = jnp.maximum(h2, 0.0)


def kernel(x1_features, x1_pos, x1_batch, x2_features, x2_pos, x2_batch,
           W1, b1, W2, b2):
    b1r = b1.reshape(1, _HID)
    b2r = b2.reshape(1, _HID)
    pp = jnp.sum(x1_pos * x1_pos, axis=1)[None, :]
    x1bf = x1_batch.astype(jnp.float32)[None, :]
    x2bf = x2_batch.astype(jnp.float32)[None, :]
    x1t = jnp.concatenate([x1_pos.T, pp, x1bf], 0)
    x2t = jnp.concatenate([x2_pos.T, x2bf], 0)

    bids = jnp.arange(_NB, dtype=jnp.int32)
    x1bi = x1_batch.astype(jnp.int32)[None, :]
    cnt = jnp.sum((x1bi == bids[:, None]).astype(jnp.int32), axis=1)
    ends = jnp.cumsum(cnt)
    starts = ends - cnt
    blo = x2_batch[0::_BQ]
    bhi = x2_batch[_BQ - 1::_BQ]
    bounds = jnp.stack([starts[blo], ends[bhi]], axis=1).astype(jnp.int32)

    u, col3 = pl.pallas_call(
        _k1_body,
        grid_spec=pltpu.PrefetchScalarGridSpec(
            num_scalar_prefetch=1,
            grid=(_NQB,),
            in_specs=[
                pl.BlockSpec((4, _BQ), lambda q, b: (0, q)),
                pl.BlockSpec((5, _N1), lambda q, b: (0, 0)),
                pl.BlockSpec((_BQ, _D), lambda q, b: (q, 0)),
                pl.BlockSpec((_D + 3, _HID), lambda q, b: (0, 0)),
                pl.BlockSpec((1, _HID), lambda q, b: (0, 0)),
            ],
            out_specs=[
                pl.BlockSpec((_BQ, _HID), lambda q, b: (q, 0)),
                pl.BlockSpec((1, 1, _BQ), lambda q, b: (q, 0, 0)),
            ],
        ),
        out_shape=[
            jax.ShapeDtypeStruct((_N1, _HID), jnp.float32),
            jax.ShapeDtypeStruct((_NQB, 1, _BQ), jnp.int32),
        ],
    )(bounds, x2t, x1t, x1_features, W1, b1r)
    col = col3.reshape(_N2)

    info = plsc.get_sparse_core_info()
    nw = info.num_cores * info.num_subcores
    bpw = _N2 // nw
    nch = bpw // 128
    col3d = col.reshape(nw, nch, 128)
    mesh = plsc.VectorSubcoreMesh(core_axis_name="c", subcore_axis_name="s")

    @functools.partial(
        pl.kernel,
        out_type=jax.ShapeDtypeStruct((_N2, _HID), jnp.float32),
        mesh=mesh,
        scratch_types=[
            pltpu.VMEM((nch, 128), jnp.int32),
            pltpu.VMEM((bpw, _HID), jnp.float32),
            pltpu.SemaphoreType.DMA,
        ],
    )
    def _sc_gather(u_hbm, idx_hbm, out_hbm, idx_v, rows_v, sem):
        w = lax.axis_index("s") * info.num_cores + lax.axis_index("c")
        pltpu.sync_copy(idx_hbm.at[w], idx_v)
        cps = [
            pltpu.async_copy(u_hbm.at[idx_v.at[j]],
                             rows_v.at[pl.ds(j * 128, 128)], sem)
            for j in range(nch)
        ]
        for cp in cps:
            cp.wait()
        pltpu.sync_copy(rows_v, out_hbm.at[pl.ds(w * bpw, bpw)])

    g = _sc_gather(u, col3d)

    out = pl.pallas_call(
        _mlp_body,
        grid=(_N2 // _BM,),
        in_specs=[
            pl.BlockSpec((_BM, _HID), lambda i: (i, 0)),
            pl.BlockSpec((4, _N2), lambda i: (0, 0)),
            pl.BlockSpec((_D + 3, _HID), lambda i: (0, 0)),
            pl.BlockSpec((_HID, _HID), lambda i: (0, 0)),
            pl.BlockSpec((1, _HID), lambda i: (0, 0)),
        ],
        out_specs=pl.BlockSpec((_BM, _HID), lambda i: (i, 0)),
        out_shape=jax.ShapeDtypeStruct((_N2, _HID), jnp.float32),
    )(g, x2t, W1, W2, b2r)

    return (out, x2_pos, x2_batch)


# speedup vs baseline: 1.1187x; 1.1187x over previous
import functools

import jax
import jax.numpy as jnp
from jax import lax
from jax.experimental import pallas as pl
from jax.experimental.pallas import tpu as pltpu
from jax.experimental.pallas import tpu_sc as plsc

_N1 = 16384
_N2 = 16384
_D = 128
_NB = 16
_HID = 128

_BM = 2048
_BQ = 256
_BK = 512
_NQB = _N2 // _BQ


def _k1_body(bounds_ref, x2t_ref, x1t_ref, xf_ref, w1_ref,
             b1_ref, u_ref, col_ref):
    q = pl.program_id(0)

    u = jnp.dot(xf_ref[...], w1_ref[0:_D, :],
                preferred_element_type=jnp.float32)
    t3 = x1t_ref[0:3, pl.ds(q * _BQ, _BQ)]
    u += lax.dot_general(t3, w1_ref[_D:, :], (((0,), (0,)), ((), ())),
                         preferred_element_type=jnp.float32)
    u_ref[...] = u + b1_ref[...]

    lo = bounds_ref[q, 0]
    hi = bounds_ref[q, 1]
    kb0 = lo // _BK
    kb1 = (hi + _BK - 1) // _BK
    t4 = x2t_ref[...]
    m2x = jnp.swapaxes(-2.0 * t4[0:1, :], 0, 1)
    m2y = jnp.swapaxes(-2.0 * t4[1:2, :], 0, 1)
    m2z = jnp.swapaxes(-2.0 * t4[2:3, :], 0, 1)
    qb = jnp.swapaxes(t4[3:4, :], 0, 1)
    inf = jnp.float32(jnp.inf)
    lanef = lax.broadcasted_iota(jnp.int32, (_BQ, _BK), 1).astype(jnp.float32)

    def one_tile(kb, valid, carry):
        bd, bl, bk = carry
        off = kb * _BK
        px = x1t_ref[0:1, pl.ds(off, _BK)]
        py = x1t_ref[1:2, pl.ds(off, _BK)]
        pz = x1t_ref[2:3, pl.ds(off, _BK)]
        ppt = x1t_ref[3:4, pl.ds(off, _BK)]
        tb = x1t_ref[4:5, pl.ds(off, _BK)]
        d = ppt + px * m2x + py * m2y + pz * m2z
        d = jnp.where(qb == tb, d, inf)
        tmin = jnp.min(d, axis=1, keepdims=True)
        cand = jnp.where(d == tmin, lanef, jnp.float32(1e9))
        targ = jnp.min(cand, axis=1, keepdims=True)
        upd = (tmin < bd) & valid
        kbf = jnp.full((_BQ, 1), kb, jnp.float32)
        return (jnp.where(upd, tmin, bd), jnp.where(upd, targ, bl),
                jnp.where(upd, kbf, bk))

    last_kb = jnp.int32(_N1 // _BK - 1)

    def tile3(i, carry):
        kb = kb0 + 3 * i
        carry = one_tile(kb, True, carry)
        kbb = jnp.minimum(kb + 1, last_kb)
        carry = one_tile(kbb, kb + 1 < kb1, carry)
        kbc = jnp.minimum(kb + 2, last_kb)
        return one_tile(kbc, kb + 2 < kb1, carry)

    bd0 = jnp.full((_BQ, 1), inf, jnp.float32)
    bl0 = jnp.zeros((_BQ, 1), jnp.float32)
    bk0 = jnp.zeros((_BQ, 1), jnp.float32)
    trips = (kb1 - kb0 + 2) // 3
    _, bl, bk = lax.fori_loop(0, trips, tile3, (bd0, bl0, bk0))
    col = (bk * float(_BK) + bl).astype(jnp.int32)
    col_ref[...] = col.T.reshape(1, 1, _BQ)


def _mlp_body(g_ref, x2t_ref, w1_ref, w2_ref, b2_ref, o_ref):
    i = pl.program_id(0)
    t3 = x2t_ref[0:3, pl.ds(i * _BM, _BM)]
    v = lax.dot_general(t3, w1_ref[_D:, :], (((0,), (0,)), ((), ())),
                        preferred_element_type=jnp.float32)
    h1 = jnp.maximum(g_ref[...] - v, 0.0)
    h2 = jnp.dot(h1, w2_ref[...], preferred_element_type=jnp.float32) + b2_ref[...]
    o_ref[...] = jnp.maximum(h2, 0.0)


def kernel(x1_features, x1_pos, x1_batch, x2_features, x2_pos, x2_batch,
           W1, b1, W2, b2):
    b1r = b1.reshape(1, _HID)
    b2r = b2.reshape(1, _HID)
    pp = jnp.sum(x1_pos * x1_pos, axis=1)[None, :]
    x1bf = x1_batch.astype(jnp.float32)[None, :]
    x2bf = x2_batch.astype(jnp.float32)[None, :]
    x1t = jnp.concatenate([x1_pos.T, pp, x1bf], 0)
    x2t = jnp.concatenate([x2_pos.T, x2bf], 0)

    bids = jnp.arange(_NB, dtype=jnp.int32)
    x1bi = x1_batch.astype(jnp.int32)[None, :]
    x2bi = x2_batch.astype(jnp.int32)[None, :]
    cnt = jnp.sum((x1bi == bids[:, None]).astype(jnp.int32), axis=1)
    ends = jnp.cumsum(cnt)
    starts = ends - cnt
    cnt2 = jnp.sum((x2bi == bids[:, None]).astype(jnp.int32), axis=1)
    ends2 = jnp.cumsum(cnt2)
    qfirst = jnp.arange(_NQB, dtype=jnp.int32)[:, None] * _BQ
    blo = jnp.sum((ends2[None, :] <= qfirst).astype(jnp.int32), axis=1)
    bhi = jnp.sum((ends2[None, :] <= qfirst + (_BQ - 1)).astype(jnp.int32), axis=1)
    bounds = jnp.stack([starts[blo], ends[bhi]], axis=1).astype(jnp.int32)

    u, col3 = pl.pallas_call(
        _k1_body,
        grid_spec=pltpu.PrefetchScalarGridSpec(
            num_scalar_prefetch=1,
            grid=(_NQB,),
            in_specs=[
                pl.BlockSpec((4, _BQ), lambda q, b: (0, q)),
                pl.BlockSpec((5, _N1), lambda q, b: (0, 0)),
                pl.BlockSpec((_BQ, _D), lambda q, b: (q, 0)),
                pl.BlockSpec((_D + 3, _HID), lambda q, b: (0, 0)),
                pl.BlockSpec((1, _HID), lambda q, b: (0, 0)),
            ],
            out_specs=[
                pl.BlockSpec((_BQ, _HID), lambda q, b: (q, 0)),
                pl.BlockSpec((1, 1, _BQ), lambda q, b: (q, 0, 0)),
            ],
        ),
        out_shape=[
            jax.ShapeDtypeStruct((_N1, _HID), jnp.float32),
            jax.ShapeDtypeStruct((_NQB, 1, _BQ), jnp.int32),
        ],
    )(bounds, x2t, x1t, x1_features, W1, b1r)
    col = col3.reshape(_N2)

    info = plsc.get_sparse_core_info()
    nw = info.num_cores * info.num_subcores
    bpw = _N2 // nw
    nch = bpw // 128
    col3d = col.reshape(nw, nch, 128)
    mesh = plsc.VectorSubcoreMesh(core_axis_name="c", subcore_axis_name="s")

    @functools.partial(
        pl.kernel,
        out_type=jax.ShapeDtypeStruct((_N2, _HID), jnp.float32),
        mesh=mesh,
        scratch_types=[
            pltpu.VMEM((nch, 128), jnp.int32),
            pltpu.VMEM((bpw, _HID), jnp.float32),
            pltpu.SemaphoreType.DMA,
        ],
    )
    def _sc_gather(u_hbm, idx_hbm, out_hbm, idx_v, rows_v, sem):
        w = lax.axis_index("s") * info.num_cores + lax.axis_index("c")
        pltpu.sync_copy(idx_hbm.at[w], idx_v)
        cps = [
            pltpu.async_copy(u_hbm.at[idx_v.at[j]],
                             rows_v.at[pl.ds(j * 128, 128)], sem)
            for j in range(nch)
        ]
        for cp in cps:
            cp.wait()
        pltpu.sync_copy(rows_v, out_hbm.at[pl.ds(w * bpw, bpw)])

    g = _sc_gather(u, col3d)

    out = pl.pallas_call(
        _mlp_body,
        grid=(_N2 // _BM,),
        in_specs=[
            pl.BlockSpec((_BM, _HID), lambda i: (i, 0)),
            pl.BlockSpec((4, _N2), lambda i: (0, 0)),
            pl.BlockSpec((_D + 3, _HID), lambda i: (0, 0)),
            pl.BlockSpec((_HID, _HID), lambda i: (0, 0)),
            pl.BlockSpec((1, _HID), lambda i: (0, 0)),
        ],
        out_specs=pl.BlockSpec((_BM, _HID), lambda i: (i, 0)),
        out_shape=jax.ShapeDtypeStruct((_N2, _HID), jnp.float32),
    )(g, x2t, W1, W2, b2r)

    return (out, x2_pos, x2_batch)
